# baseline (speedup 1.0000x reference)
import functools

import jax
import jax.numpy as jnp
from jax import lax
from jax.experimental import pallas as pl
from jax.experimental.pallas import tpu as pltpu
from jax.experimental.pallas import tpu_sc as plsc

_BATCH = 16384
_NC = 1
_NS = 16
_NW = _NC * _NS
_L = 16
_B_PER_W = _BATCH // _NW
_CHUNK = 128
_NCHUNK = _B_PER_W // _CHUNK
_NGROUP = 4

_mesh = plsc.VectorSubcoreMesh(core_axis_name="c", subcore_axis_name="s",
                               num_cores=_NC)


@functools.partial(
    pl.kernel,
    mesh=_mesh,
    out_type=jax.ShapeDtypeStruct((_NW, _NCHUNK, _CHUNK), jnp.float32),
    scratch_types=[
        pltpu.VMEM((_NCHUNK, _CHUNK), jnp.int32),
        pltpu.VMEM((_NCHUNK, _CHUNK), jnp.int32),
        pltpu.VMEM((_NCHUNK, _CHUNK), jnp.float32),
        pltpu.VMEM((_NCHUNK, _CHUNK), jnp.float32),
        pltpu.VMEM((_L,), jnp.float32),
        pltpu.SemaphoreType.DMA,
        pltpu.SemaphoreType.DMA,
        pltpu.SemaphoreType.DMA,
        pltpu.SemaphoreType.DMA,
        pltpu.SemaphoreType.DMA,
        pltpu.SemaphoreType.DMA,
    ],
)
def _mean_model_sc(uid_hbm, mid_hbm, utab_hbm, mtab_hbm, mean_hbm, out_hbm,
                   uidx_v, midx_v, u_v, m_v, mean_v,
                   sem, gsem0, gsem1, gsem2, gsem3, osem):
    wid = lax.axis_index("s") * _NC + lax.axis_index("c")
    gsems = (gsem0, gsem1, gsem2, gsem3)
    bounds = (0, 3, 5, 7, 8)

    cp_mean = pltpu.async_copy(mean_hbm, mean_v, sem)
    cp_uid = pltpu.async_copy(uid_hbm.at[wid], uidx_v, sem)
    cp_mid = pltpu.async_copy(mid_hbm.at[wid], midx_v, sem)

    cp_uid.wait()
    cp_mid.wait()
    gath = [[] for _ in range(_NGROUP)]
    for g in range(_NGROUP):
        for j in range(bounds[g], bounds[g + 1]):
            gath[g].append(
                pltpu.async_copy(utab_hbm.at[uidx_v.at[j]], u_v.at[j], gsems[g]))
            gath[g].append(
                pltpu.async_copy(mtab_hbm.at[midx_v.at[j]], m_v.at[j], gsems[g]))
    cp_mean.wait()
    mean_vec = mean_v[...]

    outs = []
    for g in range(_NGROUP):
        for c in gath[g]:
            c.wait()
        for j in range(bounds[g], bounds[g + 1]):
            for i in range(_CHUNK // _L):
                sl = pl.ds(i * _L, _L)
                u_v[j, sl] = u_v[j, sl] + m_v[j, sl] + mean_vec
        ng = bounds[g + 1] - bounds[g]
        outs.append(pltpu.async_copy(u_v.at[pl.ds(bounds[g], ng)],
                                     out_hbm.at[wid, pl.ds(bounds[g], ng)],
                                     osem))
    for c in outs:
        c.wait()


def kernel(userId, movieId, user_table, movie_table, mean):
    uid3 = userId.astype(jnp.int32).reshape(_NW, _NCHUNK, _CHUNK)
    mid3 = movieId.astype(jnp.int32).reshape(_NW, _NCHUNK, _CHUNK)
    mean16 = jnp.broadcast_to(mean.astype(jnp.float32), (_L,))
    out = _mean_model_sc(uid3, mid3, user_table, movie_table, mean16)
    return out.reshape(_BATCH)


# speedup vs baseline: 1.0070x; 1.0070x over previous
import functools

import jax
import jax.numpy as jnp
from jax import lax
from jax.experimental import pallas as pl
from jax.experimental.pallas import tpu as pltpu
from jax.experimental.pallas import tpu_sc as plsc

_BATCH = 16384
_NC = 1
_NS = 16
_NW = _NC * _NS
_L = 16
_B_PER_W = _BATCH // _NW
_CHUNK = 128
_NCHUNK = _B_PER_W // _CHUNK
_NGROUP = 4

_mesh = plsc.VectorSubcoreMesh(core_axis_name="c", subcore_axis_name="s",
                               num_cores=_NC)


@functools.partial(
    pl.kernel,
    mesh=_mesh,
    out_type=jax.ShapeDtypeStruct((_NW, _NCHUNK, _CHUNK), jnp.float32),
    scratch_types=[
        pltpu.VMEM((_NCHUNK, _CHUNK), jnp.int32),
        pltpu.VMEM((_NCHUNK, _CHUNK), jnp.int32),
        pltpu.VMEM((_NCHUNK, _CHUNK), jnp.float32),
        pltpu.VMEM((_NCHUNK, _CHUNK), jnp.float32),
        pltpu.VMEM((_L,), jnp.float32),
        pltpu.SemaphoreType.DMA,
        pltpu.SemaphoreType.DMA,
        pltpu.SemaphoreType.DMA,
        pltpu.SemaphoreType.DMA,
        pltpu.SemaphoreType.DMA,
        pltpu.SemaphoreType.DMA,
    ],
)
def _mean_model_sc(uid_hbm, mid_hbm, utab_hbm, mtab_hbm, mean_hbm, out_hbm,
                   uidx_v, midx_v, u_v, m_v, mean_v,
                   sem, gsem0, gsem1, gsem2, gsem3, osem):
    wid = lax.axis_index("s") * _NC + lax.axis_index("c")
    gsems = (gsem0, gsem1, gsem2, gsem3)
    per_g = _NCHUNK // _NGROUP

    cp_mean = pltpu.async_copy(mean_hbm, mean_v, sem)
    cp_uid = pltpu.async_copy(uid_hbm.at[wid], uidx_v, sem)
    cp_mid = pltpu.async_copy(mid_hbm.at[wid], midx_v, sem)

    cp_uid.wait()
    cp_mid.wait()
    gath = [[] for _ in range(_NGROUP)]
    for g in range(_NGROUP):
        for j in range(g * per_g, (g + 1) * per_g):
            gath[g].append(
                pltpu.async_copy(utab_hbm.at[uidx_v.at[j]], u_v.at[j], gsems[g]))
            gath[g].append(
                pltpu.async_copy(mtab_hbm.at[midx_v.at[j]], m_v.at[j], gsems[g]))
    cp_mean.wait()
    mean_vec = mean_v[...]

    outs = []
    for g in range(_NGROUP):
        for c in gath[g]:
            c.wait()
        for j in range(g * per_g, (g + 1) * per_g):
            for i in range(_CHUNK // _L):
                sl = pl.ds(i * _L, _L)
                u_v[j, sl] = u_v[j, sl] + m_v[j, sl] + mean_vec
        outs.append(pltpu.async_copy(u_v.at[pl.ds(g * per_g, per_g)],
                                     out_hbm.at[wid, pl.ds(g * per_g, per_g)],
                                     osem))
    for c in outs:
        c.wait()


def kernel(userId, movieId, user_table, movie_table, mean):
    uid3 = userId.astype(jnp.int32).reshape(_NW, _NCHUNK, _CHUNK)
    mid3 = movieId.astype(jnp.int32).reshape(_NW, _NCHUNK, _CHUNK)
    mean16 = jnp.broadcast_to(mean.astype(jnp.float32), (_L,))
    out = _mean_model_sc(uid3, mid3, user_table, movie_table, mean16)
    return out.reshape(_BATCH)
